# baseline (speedup 1.0000x reference)
import jax
import jax.numpy as jnp
from jax import lax
from jax.experimental import pallas as pl
from jax.experimental.pallas import tpu as pltpu
from jax.experimental.pallas import tpu_sc as plsc

ALPHA = 0.2
N_NODES = 10000
N_PAD = 10240
N_RELR = 200
D_IN = 128
D_OUT = 64
N_EDGES = 320000

NTILES = 16
EPT = N_EDGES // NTILES
ECH = 400
NCH = EPT // ECH
TBL = N_RELR * N_PAD
TBL_USED = 15800 * 128
SPAN_A = 988 * 128
SPAN_B = TBL_USED - 15 * SPAN_A

TILE_N = 1024
GRID_N = 10
NSLAB = TILE_N // 128


def _sc_histogram_body(types_hbm, nodes_hbm, zeros_hbm, out_hbm,
                       tb0, nb0, tb1, nb1, sb0, sb1, val, table,
                       sin0, sin1, ssc0, ssc1):
    c = lax.axis_index("c")
    w = lax.axis_index("s")
    tbufs, nbufs, sems = (tb0, tb1), (nb0, nb1), (sin0, sin1)
    sbufs, ssems = (sb0, sb1), (ssc0, ssc1)

    for b in (0, 1):
        e0 = w * EPT + b * ECH
        pltpu.async_copy(types_hbm.at[pl.ds(e0, ECH)], tbufs[b], sems[b])
        pltpu.async_copy(nodes_hbm.at[pl.ds(c * N_EDGES + e0, ECH)],
                         nbufs[b], sems[b])

    def vfill(i, carry):
        val[pl.ds(i * 16, 16)] = jnp.full((16,), 1.0, jnp.float32)
        return carry
    lax.fori_loop(0, ECH // 16, vfill, None)

    @pl.when(w < 15)
    def _zero_a():
        pltpu.sync_copy(zeros_hbm.at[pl.ds(w * SPAN_A, SPAN_A)],
                        table.at[pl.ds(w * SPAN_A, SPAN_A)])

    @pl.when(w == 15)
    def _zero_b():
        pltpu.sync_copy(zeros_hbm.at[pl.ds(15 * SPAN_A, SPAN_B)],
                        table.at[pl.ds(15 * SPAN_A, SPAN_B)])

    plsc.subcore_barrier()

    def pair(k0, carry):
        for b in (0, 1):
            k = 2 * k0 + b
            tb, nb, sin = tbufs[b], nbufs[b], sems[b]
            sb, ssc = sbufs[b], ssems[b]
            e0 = w * EPT + k * ECH
            pltpu.make_async_copy(types_hbm.at[pl.ds(e0, ECH)], tb, sin).wait()
            pltpu.make_async_copy(nodes_hbm.at[pl.ds(c * N_EDGES + e0, ECH)],
                                  nb, sin).wait()
            @pl.when(k >= 2)
            def _drain(sb=sb, ssc=ssc):
                pltpu.make_async_copy(val, table.at[sb], ssc).wait()

            def fuse(j, carry2, tb=tb, nb=nb, sb=sb):
                o = j * 16
                t = tb[pl.ds(o, 16)]
                n = nb[pl.ds(o, 16)]
                row = ((n >> 11) * 3200 + ((n >> 7) & 15) * 200 + t)
                sb[pl.ds(o, 16)] = row * 128 + (n & 127)
                return carry2
            lax.fori_loop(0, ECH // 16, fuse, None)

            pltpu.async_copy(val, table.at[sb], ssc, add=True)

            @pl.when(k + 2 < NCH)
            def _prefetch(tb=tb, nb=nb, sin=sin, k=k):
                e2 = w * EPT + (k + 2) * ECH
                pltpu.async_copy(types_hbm.at[pl.ds(e2, ECH)], tb, sin)
                pltpu.async_copy(nodes_hbm.at[pl.ds(c * N_EDGES + e2, ECH)],
                                 nb, sin)
        return carry
    lax.fori_loop(0, NCH // 2, pair, None)

    pltpu.make_async_copy(val, table.at[sb0], ssc0).wait()
    pltpu.make_async_copy(val, table.at[sb1], ssc1).wait()
    plsc.subcore_barrier()

    @pl.when(w < 15)
    def _wb_a():
        pltpu.sync_copy(table.at[pl.ds(w * SPAN_A, SPAN_A)],
                        out_hbm.at[pl.ds(c * TBL + w * SPAN_A, SPAN_A)])

    @pl.when(w == 15)
    def _wb_b():
        pltpu.sync_copy(table.at[pl.ds(15 * SPAN_A, SPAN_B)],
                        out_hbm.at[pl.ds(c * TBL + 15 * SPAN_A, SPAN_B)])


@jax.jit
def _sc_histogram(types, nodes, zeros):
    mesh = plsc.VectorSubcoreMesh(core_axis_name="c", subcore_axis_name="s")
    return pl.kernel(
        _sc_histogram_body,
        out_type=jax.ShapeDtypeStruct((2 * TBL,), jnp.float32),
        mesh=mesh,
        scratch_types=[
            pltpu.VMEM((ECH,), jnp.int32),
            pltpu.VMEM((ECH,), jnp.int32),
            pltpu.VMEM((ECH,), jnp.int32),
            pltpu.VMEM((ECH,), jnp.int32),
            pltpu.VMEM((ECH,), jnp.int32),
            pltpu.VMEM((ECH,), jnp.int32),
            pltpu.VMEM((ECH,), jnp.float32),
            pltpu.VMEM_SHARED((TBL_USED,), jnp.float32),
            pltpu.SemaphoreType.DMA,
            pltpu.SemaphoreType.DMA,
            pltpu.SemaphoreType.DMA,
            pltpu.SemaphoreType.DMA,
        ],
    )(types, nodes, zeros)


def _tc_prep_body(emb_ref, rel_ref, wrel_ref, w0_ref, w1_ref, w2_ref, w3_ref,
                  a0_ref, a1_ref, a2_ref, a3_ref,
                  ent_ref, b_out, m_out, v_out, vscr, bscr, smax):
    i = pl.program_id(0)
    w_refs = (w0_ref, w1_ref, w2_ref, w3_ref)
    a_refs = (a0_ref, a1_ref, a2_ref, a3_ref)

    @pl.when(i == 0)
    def _fold_weights():
        for br in range(4):
            wmat = w_refs[br][...]
            vscr[:, br:br + 1] = jnp.dot(wmat, a_refs[br][0:D_OUT, :],
                                         preferred_element_type=jnp.float32)
            u = jnp.dot(wmat, a_refs[br][D_OUT:2 * D_OUT, :],
                        preferred_element_type=jnp.float32)
            off = N_RELR if br < 2 else 0
            bscr[:, br:br + 1] = jnp.dot(rel_ref[off:off + N_RELR, :], u,
                                         preferred_element_type=jnp.float32)
        vscr[:, 4:8] = jnp.zeros((D_IN, 4), jnp.float32)
        bscr[:, 4:8] = jnp.zeros((N_RELR, 4), jnp.float32)
        smax[...] = jnp.zeros_like(smax)

    rows = lax.broadcasted_iota(jnp.int32, (TILE_N, D_IN), 0)
    emb = jnp.where(rows + i * TILE_N < N_NODES, emb_ref[...], 0.0)

    ent_ref[...] = jnp.dot(emb, wrel_ref[...],
                           preferred_element_type=jnp.float32)
    s = jnp.dot(emb, vscr[...], preferred_element_type=jnp.float32)
    smax[...] = jnp.maximum(smax[...], jnp.max(s, axis=0, keepdims=True))

    @pl.when(i == GRID_N - 1)
    def _emit():
        b_out[...] = bscr[...]
        v_out[...] = vscr[...]
        mm = smax[...] + bscr[...]
        m_out[...] = jnp.where(mm >= 0, mm, ALPHA * mm)


@jax.jit
def _tc_prep(entity_emb, relation_emb, W_rel, W_h0, W_h1, W_t0, W_t1,
             a_h0, a_h1, a_t0, a_t1):
    return pl.pallas_call(
        _tc_prep_body,
        grid=(GRID_N,),
        in_specs=[
            pl.BlockSpec((TILE_N, D_IN), lambda i: (i, 0)),
            pl.BlockSpec((2 * N_RELR, D_IN), lambda i: (0, 0)),
            pl.BlockSpec((D_IN, D_IN), lambda i: (0, 0)),
            pl.BlockSpec((D_IN, D_OUT), lambda i: (0, 0)),
            pl.BlockSpec((D_IN, D_OUT), lambda i: (0, 0)),
            pl.BlockSpec((D_IN, D_OUT), lambda i: (0, 0)),
            pl.BlockSpec((D_IN, D_OUT), lambda i: (0, 0)),
            pl.BlockSpec((2 * D_OUT, 1), lambda i: (0, 0)),
            pl.BlockSpec((2 * D_OUT, 1), lambda i: (0, 0)),
            pl.BlockSpec((2 * D_OUT, 1), lambda i: (0, 0)),
            pl.BlockSpec((2 * D_OUT, 1), lambda i: (0, 0)),
        ],
        out_specs=(
            pl.BlockSpec((TILE_N, D_IN), lambda i: (i, 0)),
            pl.BlockSpec((N_RELR, 8), lambda i: (0, 0)),
            pl.BlockSpec((N_RELR, 8), lambda i: (0, 0)),
            pl.BlockSpec((D_IN, 8), lambda i: (0, 0)),
        ),
        out_shape=(
            jax.ShapeDtypeStruct((N_NODES, D_IN), jnp.float32),
            jax.ShapeDtypeStruct((N_RELR, 8), jnp.float32),
            jax.ShapeDtypeStruct((N_RELR, 8), jnp.float32),
            jax.ShapeDtypeStruct((D_IN, 8), jnp.float32),
        ),
        scratch_shapes=[
            pltpu.VMEM((D_IN, 8), jnp.float32),
            pltpu.VMEM((N_RELR, 8), jnp.float32),
            pltpu.VMEM((1, 8), jnp.float32),
        ],
    )(entity_emb, relation_emb, W_rel, W_h0, W_h1, W_t0, W_t1,
      a_h0, a_h1, a_t0, a_t1)


def _tc_dense_body(nh_ref, nt_ref, emb_ref, b_ref, m_ref,
                   w0_ref, w1_ref, w2_ref, w3_ref, wr_ref, rel_ref,
                   relf_ref, accS, accd, vscr):
    i = pl.program_id(0)
    w_refs = (w0_ref, w1_ref, w2_ref, w3_ref)

    @pl.when(i == 0)
    def _init():
        accS[...] = jnp.zeros_like(accS)
        accd[...] = jnp.zeros_like(accd)

    rows = lax.broadcasted_iota(jnp.int32, (TILE_N, D_IN), 0)
    emb = jnp.where(rows + i * TILE_N < N_NODES, emb_ref[...], 0.0)
    sT = lax.dot_general(vscr[...], emb, (((0,), (1,)), ((), ())),
                         preferred_element_type=jnp.float32)

    for br in range(4):
        nref = nh_ref if br < 2 else nt_ref
        sacc = None
        dacc = None
        for tc in range(NSLAB):
            nmat = nref[pl.ds(tc * N_RELR, N_RELR), :]
            if tc == NSLAB - 1:
                cols = lax.broadcasted_iota(jnp.int32, (N_RELR, 128), 1)
                valid = i * TILE_N + tc * 128 + cols < N_NODES
                nmat = jnp.where(valid, nmat, 0.0)
            e = b_ref[:, br:br + 1] + sT[br:br + 1, tc * 128:(tc + 1) * 128]
            e = jnp.where(e >= 0, e, ALPHA * e) - m_ref[:, br:br + 1]
            ex = jnp.exp(jnp.where(e < 0, e, 0.0))
            mat = nmat * ex
            d = jnp.sum(mat, axis=1, keepdims=True)
            s = jnp.dot(mat, emb[tc * 128:(tc + 1) * 128, :],
                        preferred_element_type=jnp.float32)
            sacc = s if sacc is None else sacc + s
            dacc = d if dacc is None else dacc + d
        accd[:, br:br + 1] += dacc
        accS[br] += sacc

    @pl.when(i == GRID_N - 1)
    def _epilogue():
        outs = []
        for br in range(4):
            num = jnp.dot(accS[br], w_refs[br][...],
                          preferred_element_type=jnp.float32)
            o = num / (accd[:, br:br + 1] + 1e-16)
            outs.append(jnp.where(o > 0, o, jnp.exp(o) - 1.0))
        rr0 = outs[0] + outs[2]
        rr1 = outs[1] + outs[3]
        acc = (jnp.dot(rr0, wr_ref[0:64, :], preferred_element_type=jnp.float32)
               + jnp.dot(rr1, wr_ref[64:128, :],
                         preferred_element_type=jnp.float32))
        relproj = jnp.dot(rel_ref[...], wr_ref[128:256, :],
                          preferred_element_type=jnp.float32)
        relf_ref[...] = relproj
        relf_ref[0:200, :] = relproj[0:200, :] + acc


def _tc_dense_vscr_body(*args):
    v_ref = args[0]
    rest = args[1:]
    vscr = args[-1]
    vscr[...] = v_ref[...]
    _tc_dense_body(*rest)


@jax.jit
def _tc_dense(hist3, entity_emb, vmat, b, mstab,
              W_h0, W_h1, W_t0, W_t1, w_rel, relation_emb):
    return pl.pallas_call(
        _tc_dense_vscr_body,
        grid=(GRID_N,),
        in_specs=[
            pl.BlockSpec((D_IN, 8), lambda i: (0, 0)),
            pl.BlockSpec((NSLAB * N_RELR, 128), lambda i: (i, 0)),
            pl.BlockSpec((NSLAB * N_RELR, 128),
                         lambda i: (GRID_N + i, 0)),
            pl.BlockSpec((TILE_N, D_IN), lambda i: (i, 0)),
            pl.BlockSpec((N_RELR, 8), lambda i: (0, 0)),
            pl.BlockSpec((N_RELR, 8), lambda i: (0, 0)),
            pl.BlockSpec((D_IN, D_OUT), lambda i: (0, 0)),
            pl.BlockSpec((D_IN, D_OUT), lambda i: (0, 0)),
            pl.BlockSpec((D_IN, D_OUT), lambda i: (0, 0)),
            pl.BlockSpec((D_IN, D_OUT), lambda i: (0, 0)),
            pl.BlockSpec((2 * D_IN, D_IN), lambda i: (0, 0)),
            pl.BlockSpec((2 * N_RELR, D_IN), lambda i: (0, 0)),
        ],
        out_specs=pl.BlockSpec((2 * N_RELR, D_IN), lambda i: (0, 0)),
        out_shape=jax.ShapeDtypeStruct((2 * N_RELR, D_IN), jnp.float32),
        scratch_shapes=[
            pltpu.VMEM((4, N_RELR, D_IN), jnp.float32),
            pltpu.VMEM((N_RELR, 8), jnp.float32),
            pltpu.VMEM((D_IN, 8), jnp.float32),
        ],
    )(vmat, hist3, hist3, entity_emb, b, mstab,
      W_h0, W_h1, W_t0, W_t1, w_rel, relation_emb)


def kernel(edge_list, edge_type, entity_emb, relation_emb, W_h0, a_h0, W_h1,
           a_h1, W_t0, a_t0, W_t1, a_t1, w_rel, W_rel):
    nodes = edge_list.reshape(2 * N_EDGES)
    zeros = jnp.zeros((TBL_USED,), jnp.float32)

    ent, b, mstab, vmat = _tc_prep(entity_emb, relation_emb, W_rel,
                                   W_h0, W_h1, W_t0, W_t1,
                                   a_h0, a_h1, a_t0, a_t1)

    hist2 = _sc_histogram(edge_type, nodes, zeros).reshape(2 * TBL // 128, 128)

    rel_final = _tc_dense(hist2, entity_emb, vmat, b, mstab,
                          W_h0, W_h1, W_t0, W_t1, w_rel, relation_emb)
    return ent, rel_final


# speedup vs baseline: 1.0418x; 1.0418x over previous
import jax
import jax.numpy as jnp
from jax import lax
from jax.experimental import pallas as pl
from jax.experimental.pallas import tpu as pltpu
from jax.experimental.pallas import tpu_sc as plsc

ALPHA = 0.2
N_NODES = 10000
N_PAD = 10240
N_RELR = 200
D_IN = 128
D_OUT = 64
N_EDGES = 320000

NTILES = 16
EPT = N_EDGES // NTILES
ECH = 400
NCH = EPT // ECH
TBL = N_RELR * N_PAD
TBL_USED = 15800 * 128
SPAN_A = 988 * 128
SPAN_B = TBL_USED - 15 * SPAN_A

TILE_N = 2048
GRID_N = 5
NSLAB = TILE_N // 128


def _sc_histogram_body(types_hbm, nodes_hbm, zeros_hbm, out_hbm,
                       tb0, nb0, tb1, nb1, sb0, sb1, val, table,
                       sin0, sin1, ssc0, ssc1):
    c = lax.axis_index("c")
    w = lax.axis_index("s")
    tbufs, nbufs, sems = (tb0, tb1), (nb0, nb1), (sin0, sin1)
    sbufs, ssems = (sb0, sb1), (ssc0, ssc1)

    for b in (0, 1):
        e0 = w * EPT + b * ECH
        pltpu.async_copy(types_hbm.at[pl.ds(e0, ECH)], tbufs[b], sems[b])
        pltpu.async_copy(nodes_hbm.at[pl.ds(c * N_EDGES + e0, ECH)],
                         nbufs[b], sems[b])

    def vfill(i, carry):
        val[pl.ds(i * 16, 16)] = jnp.full((16,), 1.0, jnp.float32)
        return carry
    lax.fori_loop(0, ECH // 16, vfill, None)

    @pl.when(w < 15)
    def _zero_a():
        pltpu.sync_copy(zeros_hbm.at[pl.ds(w * SPAN_A, SPAN_A)],
                        table.at[pl.ds(w * SPAN_A, SPAN_A)])

    @pl.when(w == 15)
    def _zero_b():
        pltpu.sync_copy(zeros_hbm.at[pl.ds(15 * SPAN_A, SPAN_B)],
                        table.at[pl.ds(15 * SPAN_A, SPAN_B)])

    plsc.subcore_barrier()

    def pair(k0, carry):
        for b in (0, 1):
            k = 2 * k0 + b
            tb, nb, sin = tbufs[b], nbufs[b], sems[b]
            sb, ssc = sbufs[b], ssems[b]
            e0 = w * EPT + k * ECH
            pltpu.make_async_copy(types_hbm.at[pl.ds(e0, ECH)], tb, sin).wait()
            pltpu.make_async_copy(nodes_hbm.at[pl.ds(c * N_EDGES + e0, ECH)],
                                  nb, sin).wait()
            @pl.when(k >= 2)
            def _drain(sb=sb, ssc=ssc):
                pltpu.make_async_copy(val, table.at[sb], ssc).wait()

            def fuse(j, carry2, tb=tb, nb=nb, sb=sb):
                o = j * 16
                t = tb[pl.ds(o, 16)]
                n = nb[pl.ds(o, 16)]
                row = ((n >> 11) * 3200 + ((n >> 7) & 15) * 200 + t)
                sb[pl.ds(o, 16)] = row * 128 + (n & 127)
                return carry2
            lax.fori_loop(0, ECH // 16, fuse, None)

            pltpu.async_copy(val, table.at[sb], ssc, add=True)

            @pl.when(k + 2 < NCH)
            def _prefetch(tb=tb, nb=nb, sin=sin, k=k):
                e2 = w * EPT + (k + 2) * ECH
                pltpu.async_copy(types_hbm.at[pl.ds(e2, ECH)], tb, sin)
                pltpu.async_copy(nodes_hbm.at[pl.ds(c * N_EDGES + e2, ECH)],
                                 nb, sin)
        return carry
    lax.fori_loop(0, NCH // 2, pair, None)

    pltpu.make_async_copy(val, table.at[sb0], ssc0).wait()
    pltpu.make_async_copy(val, table.at[sb1], ssc1).wait()
    plsc.subcore_barrier()

    @pl.when(w < 15)
    def _wb_a():
        pltpu.sync_copy(table.at[pl.ds(w * SPAN_A, SPAN_A)],
                        out_hbm.at[pl.ds(c * TBL + w * SPAN_A, SPAN_A)])

    @pl.when(w == 15)
    def _wb_b():
        pltpu.sync_copy(table.at[pl.ds(15 * SPAN_A, SPAN_B)],
                        out_hbm.at[pl.ds(c * TBL + 15 * SPAN_A, SPAN_B)])


@jax.jit
def _sc_histogram(types, nodes, zeros):
    mesh = plsc.VectorSubcoreMesh(core_axis_name="c", subcore_axis_name="s")
    return pl.kernel(
        _sc_histogram_body,
        out_type=jax.ShapeDtypeStruct((2 * TBL,), jnp.float32),
        mesh=mesh,
        scratch_types=[
            pltpu.VMEM((ECH,), jnp.int32),
            pltpu.VMEM((ECH,), jnp.int32),
            pltpu.VMEM((ECH,), jnp.int32),
            pltpu.VMEM((ECH,), jnp.int32),
            pltpu.VMEM((ECH,), jnp.int32),
            pltpu.VMEM((ECH,), jnp.int32),
            pltpu.VMEM((ECH,), jnp.float32),
            pltpu.VMEM_SHARED((TBL_USED,), jnp.float32),
            pltpu.SemaphoreType.DMA,
            pltpu.SemaphoreType.DMA,
            pltpu.SemaphoreType.DMA,
            pltpu.SemaphoreType.DMA,
        ],
    )(types, nodes, zeros)


def _tc_prep_body(emb_ref, rel_ref, wrel_ref, w0_ref, w1_ref, w2_ref, w3_ref,
                  a0_ref, a1_ref, a2_ref, a3_ref,
                  ent_ref, b_out, m_out, v_out, vscr, bscr, smax):
    i = pl.program_id(0)
    w_refs = (w0_ref, w1_ref, w2_ref, w3_ref)
    a_refs = (a0_ref, a1_ref, a2_ref, a3_ref)

    @pl.when(i == 0)
    def _fold_weights():
        for br in range(4):
            wmat = w_refs[br][...]
            vscr[:, br:br + 1] = jnp.dot(wmat, a_refs[br][0:D_OUT, :],
                                         preferred_element_type=jnp.float32)
            u = jnp.dot(wmat, a_refs[br][D_OUT:2 * D_OUT, :],
                        preferred_element_type=jnp.float32)
            off = N_RELR if br < 2 else 0
            bscr[:, br:br + 1] = jnp.dot(rel_ref[off:off + N_RELR, :], u,
                                         preferred_element_type=jnp.float32)
        vscr[:, 4:8] = jnp.zeros((D_IN, 4), jnp.float32)
        bscr[:, 4:8] = jnp.zeros((N_RELR, 4), jnp.float32)
        smax[...] = jnp.zeros_like(smax)

    rows = lax.broadcasted_iota(jnp.int32, (TILE_N, D_IN), 0)
    emb = jnp.where(rows + i * TILE_N < N_NODES, emb_ref[...], 0.0)

    ent_ref[...] = jnp.dot(emb, wrel_ref[...],
                           preferred_element_type=jnp.float32)
    s = jnp.dot(emb, vscr[...], preferred_element_type=jnp.float32)
    smax[...] = jnp.maximum(smax[...], jnp.max(s, axis=0, keepdims=True))

    @pl.when(i == GRID_N - 1)
    def _emit():
        b_out[...] = bscr[...]
        v_out[...] = vscr[...]
        mm = smax[...] + bscr[...]
        m_out[...] = jnp.where(mm >= 0, mm, ALPHA * mm)


@jax.jit
def _tc_prep(entity_emb, relation_emb, W_rel, W_h0, W_h1, W_t0, W_t1,
             a_h0, a_h1, a_t0, a_t1):
    return pl.pallas_call(
        _tc_prep_body,
        grid=(GRID_N,),
        in_specs=[
            pl.BlockSpec((TILE_N, D_IN), lambda i: (i, 0)),
            pl.BlockSpec((2 * N_RELR, D_IN), lambda i: (0, 0)),
            pl.BlockSpec((D_IN, D_IN), lambda i: (0, 0)),
            pl.BlockSpec((D_IN, D_OUT), lambda i: (0, 0)),
            pl.BlockSpec((D_IN, D_OUT), lambda i: (0, 0)),
            pl.BlockSpec((D_IN, D_OUT), lambda i: (0, 0)),
            pl.BlockSpec((D_IN, D_OUT), lambda i: (0, 0)),
            pl.BlockSpec((2 * D_OUT, 1), lambda i: (0, 0)),
            pl.BlockSpec((2 * D_OUT, 1), lambda i: (0, 0)),
            pl.BlockSpec((2 * D_OUT, 1), lambda i: (0, 0)),
            pl.BlockSpec((2 * D_OUT, 1), lambda i: (0, 0)),
        ],
        out_specs=(
            pl.BlockSpec((TILE_N, D_IN), lambda i: (i, 0)),
            pl.BlockSpec((N_RELR, 8), lambda i: (0, 0)),
            pl.BlockSpec((N_RELR, 8), lambda i: (0, 0)),
            pl.BlockSpec((D_IN, 8), lambda i: (0, 0)),
        ),
        out_shape=(
            jax.ShapeDtypeStruct((N_NODES, D_IN), jnp.float32),
            jax.ShapeDtypeStruct((N_RELR, 8), jnp.float32),
            jax.ShapeDtypeStruct((N_RELR, 8), jnp.float32),
            jax.ShapeDtypeStruct((D_IN, 8), jnp.float32),
        ),
        scratch_shapes=[
            pltpu.VMEM((D_IN, 8), jnp.float32),
            pltpu.VMEM((N_RELR, 8), jnp.float32),
            pltpu.VMEM((1, 8), jnp.float32),
        ],
    )(entity_emb, relation_emb, W_rel, W_h0, W_h1, W_t0, W_t1,
      a_h0, a_h1, a_t0, a_t1)


def _tc_dense_body(nh_ref, nt_ref, emb_ref, b_ref, m_ref,
                   w0_ref, w1_ref, w2_ref, w3_ref, wr_ref, rel_ref,
                   relf_ref, accS, accd, vscr):
    i = pl.program_id(0)
    w_refs = (w0_ref, w1_ref, w2_ref, w3_ref)

    @pl.when(i == 0)
    def _init():
        accS[...] = jnp.zeros_like(accS)
        accd[...] = jnp.zeros_like(accd)

    rows = lax.broadcasted_iota(jnp.int32, (TILE_N, D_IN), 0)
    emb = jnp.where(rows + i * TILE_N < N_NODES, emb_ref[...], 0.0)
    sT = lax.dot_general(vscr[...], emb, (((0,), (1,)), ((), ())),
                         preferred_element_type=jnp.float32)

    for br in range(4):
        nref = nh_ref if br < 2 else nt_ref
        sacc = None
        dacc = None
        for tc in range(NSLAB):
            nmat = nref[pl.ds(tc * N_RELR, N_RELR), :]
            if tc == NSLAB - 1:
                cols = lax.broadcasted_iota(jnp.int32, (N_RELR, 128), 1)
                valid = i * TILE_N + tc * 128 + cols < N_NODES
                nmat = jnp.where(valid, nmat, 0.0)
            e = b_ref[:, br:br + 1] + sT[br:br + 1, tc * 128:(tc + 1) * 128]
            e = jnp.where(e >= 0, e, ALPHA * e) - m_ref[:, br:br + 1]
            ex = jnp.exp(jnp.where(e < 0, e, 0.0))
            mat = nmat * ex
            d = jnp.sum(mat, axis=1, keepdims=True)
            s = jnp.dot(mat, emb[tc * 128:(tc + 1) * 128, :],
                        preferred_element_type=jnp.float32)
            sacc = s if sacc is None else sacc + s
            dacc = d if dacc is None else dacc + d
        accd[:, br:br + 1] += dacc
        accS[br] += sacc

    @pl.when(i == GRID_N - 1)
    def _epilogue():
        outs = []
        for br in range(4):
            num = jnp.dot(accS[br], w_refs[br][...],
                          preferred_element_type=jnp.float32)
            o = num / (accd[:, br:br + 1] + 1e-16)
            outs.append(jnp.where(o > 0, o, jnp.exp(o) - 1.0))
        rr0 = outs[0] + outs[2]
        rr1 = outs[1] + outs[3]
        acc = (jnp.dot(rr0, wr_ref[0:64, :], preferred_element_type=jnp.float32)
               + jnp.dot(rr1, wr_ref[64:128, :],
                         preferred_element_type=jnp.float32))
        relproj = jnp.dot(rel_ref[...], wr_ref[128:256, :],
                          preferred_element_type=jnp.float32)
        relf_ref[...] = relproj
        relf_ref[0:200, :] = relproj[0:200, :] + acc


def _tc_dense_vscr_body(*args):
    v_ref = args[0]
    rest = args[1:]
    vscr = args[-1]
    vscr[...] = v_ref[...]
    _tc_dense_body(*rest)


@jax.jit
def _tc_dense(hist3, entity_emb, vmat, b, mstab,
              W_h0, W_h1, W_t0, W_t1, w_rel, relation_emb):
    return pl.pallas_call(
        _tc_dense_vscr_body,
        grid=(GRID_N,),
        in_specs=[
            pl.BlockSpec((D_IN, 8), lambda i: (0, 0)),
            pl.BlockSpec((NSLAB * N_RELR, 128), lambda i: (i, 0)),
            pl.BlockSpec((NSLAB * N_RELR, 128),
                         lambda i: (GRID_N + i, 0)),
            pl.BlockSpec((TILE_N, D_IN), lambda i: (i, 0)),
            pl.BlockSpec((N_RELR, 8), lambda i: (0, 0)),
            pl.BlockSpec((N_RELR, 8), lambda i: (0, 0)),
            pl.BlockSpec((D_IN, D_OUT), lambda i: (0, 0)),
            pl.BlockSpec((D_IN, D_OUT), lambda i: (0, 0)),
            pl.BlockSpec((D_IN, D_OUT), lambda i: (0, 0)),
            pl.BlockSpec((D_IN, D_OUT), lambda i: (0, 0)),
            pl.BlockSpec((2 * D_IN, D_IN), lambda i: (0, 0)),
            pl.BlockSpec((2 * N_RELR, D_IN), lambda i: (0, 0)),
        ],
        out_specs=pl.BlockSpec((2 * N_RELR, D_IN), lambda i: (0, 0)),
        out_shape=jax.ShapeDtypeStruct((2 * N_RELR, D_IN), jnp.float32),
        scratch_shapes=[
            pltpu.VMEM((4, N_RELR, D_IN), jnp.float32),
            pltpu.VMEM((N_RELR, 8), jnp.float32),
            pltpu.VMEM((D_IN, 8), jnp.float32),
        ],
    )(vmat, hist3, hist3, entity_emb, b, mstab,
      W_h0, W_h1, W_t0, W_t1, w_rel, relation_emb)


def kernel(edge_list, edge_type, entity_emb, relation_emb, W_h0, a_h0, W_h1,
           a_h1, W_t0, a_t0, W_t1, a_t1, w_rel, W_rel):
    nodes = edge_list.reshape(2 * N_EDGES)
    zeros = jnp.zeros((TBL_USED,), jnp.float32)

    ent, b, mstab, vmat = _tc_prep(entity_emb, relation_emb, W_rel,
                                   W_h0, W_h1, W_t0, W_t1,
                                   a_h0, a_h1, a_t0, a_t1)

    hist2 = _sc_histogram(edge_type, nodes, zeros).reshape(2 * TBL // 128, 128)

    rel_final = _tc_dense(hist2, entity_emb, vmat, b, mstab,
                          W_h0, W_h1, W_t0, W_t1, w_rel, relation_emb)
    return ent, rel_final
